# Initial kernel scaffold; baseline (speedup 1.0000x reference)
import jax
import jax.numpy as jnp
from jax.experimental import pallas as pl


def kernel(particles, particles_probs):
    raise NotImplementedError("write your pallas kernel here")


# speedup vs baseline: 1.0136x; 1.0136x over previous
import functools

import jax
import jax.numpy as jnp
from jax import lax
from jax.experimental import pallas as pl
from jax.experimental.pallas import tpu as pltpu
from jax.experimental.pallas import tpu_sc as plsc

N = 65536
D = 32
NC = 2
NS = 16
NW = NC * NS
B_PER_W = N // NW
CHUNK = 128
NCHUNK = B_PER_W // CHUNK


def _gather_body(table_hbm, idx_hbm, out_hbm, idx_v, rows_v, sem):
    wid = lax.axis_index("s") * NC + lax.axis_index("c")
    base = wid * B_PER_W
    pltpu.sync_copy(idx_hbm.at[wid], idx_v)
    copies = []
    for j in range(NCHUNK):
        copies.append(
            pltpu.async_copy(
                table_hbm.at[idx_v.at[j]],
                rows_v.at[pl.ds(j * CHUNK, CHUNK)],
                sem,
            )
        )
    for c in copies:
        c.wait()
    pltpu.sync_copy(rows_v, out_hbm.at[pl.ds(base, B_PER_W)])


@functools.partial(jax.jit, static_argnames=())
def _sc_gather(particles, idx):
    idx3 = idx.reshape(NW, NCHUNK, CHUNK)
    run = pl.kernel(
        _gather_body,
        out_type=jax.ShapeDtypeStruct((N, D), jnp.float32),
        mesh=plsc.VectorSubcoreMesh(core_axis_name="c", subcore_axis_name="s"),
        scratch_types=[
            pltpu.VMEM((NCHUNK, CHUNK), jnp.int32),
            pltpu.VMEM((B_PER_W, D), jnp.float32),
            pltpu.SemaphoreType.DMA,
        ],
        compiler_params=pltpu.CompilerParams(use_tc_tiling_on_sc=False),
    )
    return run(particles, idx3)


def kernel(particles, particles_probs):
    n = particles.shape[0]
    step = 1.0 / n
    probs = particles_probs / jnp.sum(particles_probs)
    rnd_offset = jax.random.uniform(jax.random.key(42), (), dtype=jnp.float32,
                                    minval=0.0, maxval=step)
    positions = rnd_offset + step * jnp.arange(n, dtype=jnp.float32)
    cum = jnp.cumsum(probs)
    idx = jnp.searchsorted(cum, positions, side='left')
    idx = jnp.clip(idx, 0, n - 1).astype(jnp.int32)
    return _sc_gather(particles, idx)


# speedup vs baseline: 10.0410x; 9.9061x over previous
import jax
import jax.numpy as jnp
from jax import lax
from jax.experimental import pallas as pl
from jax.experimental.pallas import tpu as pltpu
from jax.experimental.pallas import tpu_sc as plsc

N = 65536
D = 32
STEP = jnp.float32(1.0 / N)
NC = 2
NS = 16
NW = NC * NS
B_PER_W = N // NW
L = 16
CHUNK = 128
HALF = B_PER_W // 2


def _resample_body(cum_hbm, off_hbm, table_hbm, out_hbm,
                   cum_v, off_v, idx_v, rows_v, sem):
    wid = lax.axis_index("s") * NC + lax.axis_index("c")
    base = wid * B_PER_W

    pltpu.sync_copy(cum_hbm, cum_v)
    pltpu.sync_copy(off_hbm, off_v)
    off = off_v[...]
    lanes = lax.iota(jnp.int32, L)

    def chunk_body(c, carry):
        jv = base + c * L + lanes
        pos = off + STEP * jv.astype(jnp.float32)
        r = jnp.zeros((L,), jnp.int32)
        s = 1 << 15
        while s >= 1:
            t = r + s
            cm = plsc.load_gather(cum_v, [t - 1])
            r = jnp.where(cm < pos, t, r)
            s >>= 1
        idx_v[pl.ds(c * L, L)] = jnp.minimum(r, N - 1)
        return carry

    lax.fori_loop(0, B_PER_W // L, chunk_body, 0)

    for h in range(2):
        copies = []
        for j in range(HALF // CHUNK):
            copies.append(
                pltpu.async_copy(
                    table_hbm.at[idx_v.at[pl.ds(h * HALF + j * CHUNK, CHUNK)]],
                    rows_v.at[pl.ds(j * CHUNK, CHUNK)],
                    sem,
                )
            )
        for c in copies:
            c.wait()
        pltpu.sync_copy(rows_v, out_hbm.at[pl.ds(base + h * HALF, HALF)])


def _sc_resample(cum, off_arr, particles):
    run = pl.kernel(
        _resample_body,
        out_type=jax.ShapeDtypeStruct((N, D), jnp.float32),
        mesh=plsc.VectorSubcoreMesh(core_axis_name="c", subcore_axis_name="s"),
        scratch_types=[
            pltpu.VMEM((N,), jnp.float32),
            pltpu.VMEM((L,), jnp.float32),
            pltpu.VMEM((B_PER_W,), jnp.int32),
            pltpu.VMEM((HALF, D), jnp.float32),
            pltpu.SemaphoreType.DMA,
        ],
        compiler_params=pltpu.CompilerParams(use_tc_tiling_on_sc=False,
                                             needs_layout_passes=False),
    )
    return run(cum, off_arr, particles)


def kernel(particles, particles_probs):
    n = particles.shape[0]
    probs = particles_probs / jnp.sum(particles_probs)
    cum = jnp.cumsum(probs)
    rnd_offset = jax.random.uniform(jax.random.key(42), (), dtype=jnp.float32,
                                    minval=0.0, maxval=1.0 / n)
    off_arr = jnp.full((L,), rnd_offset, dtype=jnp.float32)
    return _sc_resample(cum, off_arr, particles)
